# baseline (speedup 1.0000x reference)
import functools

import jax
import jax.numpy as jnp
from jax import lax
from jax.experimental import pallas as pl
from jax.experimental.pallas import tpu as pltpu
from jax.experimental.pallas import tpu_sc as plsc
import numpy as np

B, E_DIM, L = 8, 384, 576
N_E, H_DIM = 8192, 64
BETA = 0.25
EPS = float(np.finfo(np.float32).eps)
T = B * L

KB = 8192
NKB = N_E // KB
SUB = 1024

NW = 32
RPW = T // NW
RH = RPW // 2


def _sim_argmax_body(z_ref, w_in_ref, b_in_ref, cb_ref, w_out_ref, b_out_ref,
                     inds_ref, loss_ref, tbl_ref,
                     zn_s, mx_s, ix_s, acc):
    k = pl.program_id(0)

    @pl.when(k == 0)
    def _():
        zns = []
        for bb in range(B):
            zp = jnp.dot(w_in_ref[...], z_ref[bb],
                         preferred_element_type=jnp.float32) + b_in_ref[...]
            ms = jnp.mean(zp * zp, axis=0, keepdims=True)
            zns.append(zp * lax.rsqrt(ms + EPS))
        zn_all = jnp.concatenate(zns, axis=1)
        zn_s[...] = zn_all
        acc[0, 0] = jnp.sum(zn_all * zn_all)
        mx_s[...] = jnp.full((1, T), -jnp.inf, jnp.float32)
        ix_s[...] = jnp.zeros((1, T), jnp.int32)

    cb = cb_ref[...]
    cms = jnp.mean(cb * cb, axis=1, keepdims=True)
    cbn = cb * lax.rsqrt(cms + EPS)
    tbl_ref[...] = lax.dot_general(
        cbn, w_out_ref[...], (((1,), (1,)), ((), ())),
        preferred_element_type=jnp.float32) + b_out_ref[...]

    zn = zn_s[...]
    m = mx_s[...]
    li = ix_s[...]
    for j in range(KB // SUB):
        sj = jnp.dot(cbn[j * SUB:(j + 1) * SUB, :], zn,
                     preferred_element_type=jnp.float32)
        mj = jnp.max(sj, axis=0, keepdims=True)
        rows = lax.broadcasted_iota(jnp.int32, sj.shape, 0).astype(jnp.float32)
        ljf = jnp.min(jnp.where(sj == mj, rows, jnp.float32(SUB)),
                      axis=0, keepdims=True)
        lj = ljf.astype(jnp.int32) + (k * KB + j * SUB)
        upd = mj > m
        li = jnp.where(upd, lj, li)
        m = jnp.where(upd, mj, m)
    ix_s[...] = li
    mx_s[...] = m

    @pl.when(k == pl.num_programs(0) - 1)
    def _():
        inds_ref[...] = li
        loss_ref[0, 0] = ((acc[0, 0] - 2.0 * jnp.sum(m)
                           + jnp.float32(T * H_DIM))
                          * ((1.0 + BETA) / float(T * H_DIM)))


_sim_argmax = pl.pallas_call(
    _sim_argmax_body,
    grid=(NKB,),
    in_specs=[
        pl.BlockSpec((B, E_DIM, L), lambda k: (0, 0, 0)),
        pl.BlockSpec((H_DIM, E_DIM), lambda k: (0, 0)),
        pl.BlockSpec((H_DIM, 1), lambda k: (0, 0)),
        pl.BlockSpec((KB, H_DIM), lambda k: (k, 0)),
        pl.BlockSpec((E_DIM, H_DIM), lambda k: (0, 0)),
        pl.BlockSpec((1, E_DIM), lambda k: (0, 0)),
    ],
    out_specs=[
        pl.BlockSpec((1, T), lambda k: (0, 0)),
        pl.BlockSpec(memory_space=pltpu.SMEM),
        pl.BlockSpec((KB, E_DIM), lambda k: (k, 0)),
    ],
    out_shape=[
        jax.ShapeDtypeStruct((1, T), jnp.int32),
        jax.ShapeDtypeStruct((1, 1), jnp.float32),
        jax.ShapeDtypeStruct((N_E, E_DIM), jnp.float32),
    ],
    scratch_shapes=[
        pltpu.VMEM((H_DIM, T), jnp.float32),
        pltpu.VMEM((1, T), jnp.float32),
        pltpu.VMEM((1, T), jnp.int32),
        pltpu.SMEM((1, 1), jnp.float32),
    ],
)


@functools.cache
def _make_sc_gather():
    mesh = plsc.VectorSubcoreMesh(core_axis_name="c", subcore_axis_name="s")

    @functools.partial(
        pl.kernel, mesh=mesh,
        out_type=jax.ShapeDtypeStruct((T, E_DIM), jnp.float32),
        scratch_types=[
            pltpu.VMEM((RH,), jnp.int32),
            pltpu.VMEM((RH,), jnp.int32),
            pltpu.VMEM((RPW, E_DIM), jnp.float32),
            pltpu.SemaphoreType.DMA,
            pltpu.SemaphoreType.DMA,
            pltpu.SemaphoreType.DMA,
        ],
    )
    def gather_k(table_hbm, idx_hbm, out_hbm, idx_a, idx_b, rows_v,
                 sem_a, sem_b, sem_o):
        wid = lax.axis_index("s") * 2 + lax.axis_index("c")
        base = wid * RPW
        pltpu.sync_copy(idx_hbm.at[pl.ds(base, RH)], idx_a)
        pltpu.sync_copy(idx_hbm.at[pl.ds(base + RH, RH)], idx_b)
        c1 = pltpu.async_copy(table_hbm.at[idx_a],
                              rows_v.at[pl.ds(0, RH)], sem_a)
        c2 = pltpu.async_copy(table_hbm.at[idx_b],
                              rows_v.at[pl.ds(RH, RH)], sem_b)
        c1.wait()
        o1 = pltpu.async_copy(rows_v.at[pl.ds(0, RH)],
                              out_hbm.at[pl.ds(base, RH)], sem_o)
        c2.wait()
        o2 = pltpu.async_copy(rows_v.at[pl.ds(RH, RH)],
                              out_hbm.at[pl.ds(base + RH, RH)], sem_o)
        o1.wait()
        o2.wait()

    return gather_k


def kernel(z, W_in, b_in, codebook, W_out, b_out):
    inds2, loss, proj_tbl = _sim_argmax(z, W_in, b_in.reshape(H_DIM, 1),
                                        codebook, W_out,
                                        b_out.reshape(1, E_DIM))
    inds = inds2.reshape(B, L)
    out_tok = _make_sc_gather()(proj_tbl, inds.reshape(T))
    out = jnp.transpose(out_tok.reshape(B, L, E_DIM), (0, 2, 1))
    return out, inds, loss.reshape(())


# speedup vs baseline: 1.0268x; 1.0268x over previous
import functools

import jax
import jax.numpy as jnp
from jax import lax
from jax.experimental import pallas as pl
from jax.experimental.pallas import tpu as pltpu
from jax.experimental.pallas import tpu_sc as plsc
import numpy as np

B, E_DIM, L = 8, 384, 576
N_E, H_DIM = 8192, 64
BETA = 0.25
EPS = float(np.finfo(np.float32).eps)
T = B * L

KB = 8192
NKB = N_E // KB
SUB = 256

NW = 32
RPW = T // NW
RH = RPW // 2


def _sim_argmax_body(z_ref, w_in_ref, b_in_ref, cb_ref, w_out_ref, b_out_ref,
                     inds_ref, loss_ref, tbl_ref,
                     zn_s, mx_s, ix_s, acc):
    k = pl.program_id(0)

    @pl.when(k == 0)
    def _():
        zns = []
        for bb in range(B):
            zp = jnp.dot(w_in_ref[...], z_ref[bb],
                         preferred_element_type=jnp.float32) + b_in_ref[...]
            ms = jnp.mean(zp * zp, axis=0, keepdims=True)
            zns.append(zp * lax.rsqrt(ms + EPS))
        zn_all = jnp.concatenate(zns, axis=1)
        zn_s[...] = zn_all
        acc[0, 0] = jnp.sum(zn_all * zn_all)
        mx_s[...] = jnp.full((1, T), -jnp.inf, jnp.float32)
        ix_s[...] = jnp.zeros((1, T), jnp.int32)

    cb = cb_ref[...]
    cms = jnp.mean(cb * cb, axis=1, keepdims=True)
    cbn = cb * lax.rsqrt(cms + EPS)
    tbl_ref[...] = lax.dot_general(
        cbn, w_out_ref[...], (((1,), (1,)), ((), ())),
        preferred_element_type=jnp.float32) + b_out_ref[...]

    zn = zn_s[...]
    m = mx_s[...]
    li = ix_s[...]
    for j in range(KB // SUB):
        sj = jnp.dot(cbn[j * SUB:(j + 1) * SUB, :], zn,
                     preferred_element_type=jnp.float32)
        mj = jnp.max(sj, axis=0, keepdims=True)
        rows = lax.broadcasted_iota(jnp.int32, sj.shape, 0).astype(jnp.float32)
        ljf = jnp.min(jnp.where(sj == mj, rows, jnp.float32(SUB)),
                      axis=0, keepdims=True)
        lj = ljf.astype(jnp.int32) + (k * KB + j * SUB)
        upd = mj > m
        li = jnp.where(upd, lj, li)
        m = jnp.where(upd, mj, m)
    ix_s[...] = li
    mx_s[...] = m

    @pl.when(k == pl.num_programs(0) - 1)
    def _():
        inds_ref[...] = li
        loss_ref[0, 0] = ((acc[0, 0] - 2.0 * jnp.sum(m)
                           + jnp.float32(T * H_DIM))
                          * ((1.0 + BETA) / float(T * H_DIM)))


_sim_argmax = pl.pallas_call(
    _sim_argmax_body,
    grid=(NKB,),
    in_specs=[
        pl.BlockSpec((B, E_DIM, L), lambda k: (0, 0, 0)),
        pl.BlockSpec((H_DIM, E_DIM), lambda k: (0, 0)),
        pl.BlockSpec((H_DIM, 1), lambda k: (0, 0)),
        pl.BlockSpec((KB, H_DIM), lambda k: (k, 0)),
        pl.BlockSpec((E_DIM, H_DIM), lambda k: (0, 0)),
        pl.BlockSpec((1, E_DIM), lambda k: (0, 0)),
    ],
    out_specs=[
        pl.BlockSpec((1, T), lambda k: (0, 0)),
        pl.BlockSpec(memory_space=pltpu.SMEM),
        pl.BlockSpec((KB, E_DIM), lambda k: (k, 0)),
    ],
    out_shape=[
        jax.ShapeDtypeStruct((1, T), jnp.int32),
        jax.ShapeDtypeStruct((1, 1), jnp.float32),
        jax.ShapeDtypeStruct((N_E, E_DIM), jnp.float32),
    ],
    scratch_shapes=[
        pltpu.VMEM((H_DIM, T), jnp.float32),
        pltpu.VMEM((1, T), jnp.float32),
        pltpu.VMEM((1, T), jnp.int32),
        pltpu.SMEM((1, 1), jnp.float32),
    ],
)


@functools.cache
def _make_sc_gather():
    mesh = plsc.VectorSubcoreMesh(core_axis_name="c", subcore_axis_name="s")

    @functools.partial(
        pl.kernel, mesh=mesh,
        out_type=jax.ShapeDtypeStruct((T, E_DIM), jnp.float32),
        scratch_types=[
            pltpu.VMEM((RH,), jnp.int32),
            pltpu.VMEM((RH,), jnp.int32),
            pltpu.VMEM((RPW, E_DIM), jnp.float32),
            pltpu.SemaphoreType.DMA,
            pltpu.SemaphoreType.DMA,
            pltpu.SemaphoreType.DMA,
        ],
    )
    def gather_k(table_hbm, idx_hbm, out_hbm, idx_a, idx_b, rows_v,
                 sem_a, sem_b, sem_o):
        wid = lax.axis_index("s") * 2 + lax.axis_index("c")
        base = wid * RPW
        pltpu.sync_copy(idx_hbm.at[pl.ds(base, RH)], idx_a)
        pltpu.sync_copy(idx_hbm.at[pl.ds(base + RH, RH)], idx_b)
        c1 = pltpu.async_copy(table_hbm.at[idx_a],
                              rows_v.at[pl.ds(0, RH)], sem_a)
        c2 = pltpu.async_copy(table_hbm.at[idx_b],
                              rows_v.at[pl.ds(RH, RH)], sem_b)
        c1.wait()
        o1 = pltpu.async_copy(rows_v.at[pl.ds(0, RH)],
                              out_hbm.at[pl.ds(base, RH)], sem_o)
        c2.wait()
        o2 = pltpu.async_copy(rows_v.at[pl.ds(RH, RH)],
                              out_hbm.at[pl.ds(base + RH, RH)], sem_o)
        o1.wait()
        o2.wait()

    return gather_k


def kernel(z, W_in, b_in, codebook, W_out, b_out):
    inds2, loss, proj_tbl = _sim_argmax(z, W_in, b_in.reshape(H_DIM, 1),
                                        codebook, W_out,
                                        b_out.reshape(1, E_DIM))
    inds = inds2.reshape(B, L)
    out_tok = _make_sc_gather()(proj_tbl, inds.reshape(T))
    out = jnp.transpose(out_tok.reshape(B, L, E_DIM), (0, 2, 1))
    return out, inds, loss.reshape(())


# speedup vs baseline: 1.0403x; 1.0131x over previous
import functools

import jax
import jax.numpy as jnp
from jax import lax
from jax.experimental import pallas as pl
from jax.experimental.pallas import tpu as pltpu
from jax.experimental.pallas import tpu_sc as plsc
import numpy as np

B, E_DIM, L = 8, 384, 576
N_E, H_DIM = 8192, 64
BETA = 0.25
EPS = float(np.finfo(np.float32).eps)
T = B * L

KB = 8192
NKB = N_E // KB
SUB = 512

NW = 32
RPW = T // NW
RH = RPW // 2


def _sim_argmax_body(z_ref, w_in_ref, b_in_ref, cb_ref, w_out_ref, b_out_ref,
                     inds_ref, loss_ref, tbl_ref,
                     zn_s, mx_s, ix_s, acc):
    k = pl.program_id(0)

    @pl.when(k == 0)
    def _():
        zns = []
        for bb in range(B):
            zp = jnp.dot(w_in_ref[...], z_ref[bb],
                         preferred_element_type=jnp.float32) + b_in_ref[...]
            ms = jnp.mean(zp * zp, axis=0, keepdims=True)
            zns.append(zp * lax.rsqrt(ms + EPS))
        zn_all = jnp.concatenate(zns, axis=1)
        zn_s[...] = zn_all
        acc[0, 0] = jnp.sum(zn_all * zn_all)
        mx_s[...] = jnp.full((1, T), -jnp.inf, jnp.float32)
        ix_s[...] = jnp.zeros((1, T), jnp.int32)

    cb = cb_ref[...]
    cms = jnp.mean(cb * cb, axis=1, keepdims=True)
    cbn = cb * lax.rsqrt(cms + EPS)
    tbl_ref[...] = lax.dot_general(
        cbn, w_out_ref[...], (((1,), (1,)), ((), ())),
        preferred_element_type=jnp.float32) + b_out_ref[...]

    zn = zn_s[...]
    m = mx_s[...]
    li = ix_s[...]
    for j in range(KB // SUB):
        sj = jnp.dot(cbn[j * SUB:(j + 1) * SUB, :], zn,
                     preferred_element_type=jnp.float32)
        mj = jnp.max(sj, axis=0, keepdims=True)
        rows = lax.broadcasted_iota(jnp.int32, sj.shape, 0).astype(jnp.float32)
        ljf = jnp.min(jnp.where(sj == mj, rows, jnp.float32(SUB)),
                      axis=0, keepdims=True)
        lj = ljf.astype(jnp.int32) + (k * KB + j * SUB)
        upd = mj > m
        li = jnp.where(upd, lj, li)
        m = jnp.where(upd, mj, m)
    ix_s[...] = li
    mx_s[...] = m

    @pl.when(k == pl.num_programs(0) - 1)
    def _():
        inds_ref[...] = li
        loss_ref[0, 0] = ((acc[0, 0] - 2.0 * jnp.sum(m)
                           + jnp.float32(T * H_DIM))
                          * ((1.0 + BETA) / float(T * H_DIM)))


_sim_argmax = pl.pallas_call(
    _sim_argmax_body,
    grid=(NKB,),
    in_specs=[
        pl.BlockSpec((B, E_DIM, L), lambda k: (0, 0, 0)),
        pl.BlockSpec((H_DIM, E_DIM), lambda k: (0, 0)),
        pl.BlockSpec((H_DIM, 1), lambda k: (0, 0)),
        pl.BlockSpec((KB, H_DIM), lambda k: (k, 0)),
        pl.BlockSpec((E_DIM, H_DIM), lambda k: (0, 0)),
        pl.BlockSpec((1, E_DIM), lambda k: (0, 0)),
    ],
    out_specs=[
        pl.BlockSpec((1, T), lambda k: (0, 0)),
        pl.BlockSpec(memory_space=pltpu.SMEM),
        pl.BlockSpec((KB, E_DIM), lambda k: (k, 0)),
    ],
    out_shape=[
        jax.ShapeDtypeStruct((1, T), jnp.int32),
        jax.ShapeDtypeStruct((1, 1), jnp.float32),
        jax.ShapeDtypeStruct((N_E, E_DIM), jnp.float32),
    ],
    scratch_shapes=[
        pltpu.VMEM((H_DIM, T), jnp.float32),
        pltpu.VMEM((1, T), jnp.float32),
        pltpu.VMEM((1, T), jnp.int32),
        pltpu.SMEM((1, 1), jnp.float32),
    ],
)


@functools.cache
def _make_sc_gather():
    mesh = plsc.VectorSubcoreMesh(core_axis_name="c", subcore_axis_name="s")

    @functools.partial(
        pl.kernel, mesh=mesh,
        out_type=jax.ShapeDtypeStruct((T, E_DIM), jnp.float32),
        scratch_types=[
            pltpu.VMEM((RH,), jnp.int32),
            pltpu.VMEM((RH,), jnp.int32),
            pltpu.VMEM((RPW, E_DIM), jnp.float32),
            pltpu.SemaphoreType.DMA,
            pltpu.SemaphoreType.DMA,
            pltpu.SemaphoreType.DMA,
        ],
    )
    def gather_k(table_hbm, idx_hbm, out_hbm, idx_a, idx_b, rows_v,
                 sem_a, sem_b, sem_o):
        wid = lax.axis_index("s") * 2 + lax.axis_index("c")
        base = wid * RPW
        pltpu.sync_copy(idx_hbm.at[pl.ds(base, RH)], idx_a)
        pltpu.sync_copy(idx_hbm.at[pl.ds(base + RH, RH)], idx_b)
        c1 = pltpu.async_copy(table_hbm.at[idx_a],
                              rows_v.at[pl.ds(0, RH)], sem_a)
        c2 = pltpu.async_copy(table_hbm.at[idx_b],
                              rows_v.at[pl.ds(RH, RH)], sem_b)
        c1.wait()
        o1 = pltpu.async_copy(rows_v.at[pl.ds(0, RH)],
                              out_hbm.at[pl.ds(base, RH)], sem_o)
        c2.wait()
        o2 = pltpu.async_copy(rows_v.at[pl.ds(RH, RH)],
                              out_hbm.at[pl.ds(base + RH, RH)], sem_o)
        o1.wait()
        o2.wait()

    return gather_k


def kernel(z, W_in, b_in, codebook, W_out, b_out):
    inds2, loss, proj_tbl = _sim_argmax(z, W_in, b_in.reshape(H_DIM, 1),
                                        codebook, W_out,
                                        b_out.reshape(1, E_DIM))
    inds = inds2.reshape(B, L)
    out_tok = _make_sc_gather()(proj_tbl, inds.reshape(T))
    out = jnp.transpose(out_tok.reshape(B, L, E_DIM), (0, 2, 1))
    return out, inds, loss.reshape(())
